# baseline (speedup 1.0000x reference)
import jax
import jax.numpy as jnp
from jax import lax
from jax.experimental import pallas as pl
from jax.experimental.pallas import tpu as pltpu
from jax.experimental.pallas import tpu_sc as plsc

N = 10000
D = 128
H = 256
E = 160000
M = 16384
NCORES = 2
NSUB = 16
NW = NCORES * NSUB
CH = 128
EP = 163840
ECH = EP // CH
NCH1 = ECH // NW
NCH2 = ECH // NSUB
NB = 2
SR = 40
NPAD = 10240
RPT = NPAD // NSUB
OPW = M // NW

_mesh = plsc.VectorSubcoreMesh(core_axis_name="c", subcore_axis_name="s")


def _make_scatter_body(nch, split_by_wid):
    halves = nch // SR

    def body(ha_hbm, hb_hbm, srcr_hbm, dstr_hbm, out_hbm,
             sidx, didx, rows, acc, isem, g0, g1, s0, s1):
        c = lax.axis_index("c")
        s = lax.axis_index("s")
        wid = s * NCORES + c
        rowbase = (wid if split_by_wid else s) * nch

        d_si = pltpu.async_copy(srcr_hbm.at[pl.ds(rowbase, SR)], sidx, isem)
        d_di = pltpu.async_copy(dstr_hbm.at[pl.ds(rowbase, SR)], didx, isem)

        z16 = jnp.zeros((16,), jnp.float32)

        def _zrow(r, _):
            def _zcol(cc, _):
                rows[0, r, pl.ds(cc * 16, 16)] = z16
                return 0
            return lax.fori_loop(0, D // 16, _zcol, 0)

        lax.fori_loop(0, CH, _zrow, 0)

        def _zacc(j, _):
            pltpu.sync_copy(rows.at[0], acc.at[pl.ds(s * RPT + j * CH, CH)])
            return 0

        lax.fori_loop(0, RPT // CH, _zacc, 0)
        d_si.wait()
        d_di.wait()

        gsems = (g0, g1)
        ssems = (s0, s1)

        def _gather(j, b, sem):
            srow = sidx.at[j]

            @pl.when(c == 0)
            def _():
                pltpu.async_copy(ha_hbm.at[srow], rows.at[b], sem)

            @pl.when(c == 1)
            def _():
                pltpu.async_copy(hb_hbm.at[srow], rows.at[b], sem)

        def _wait_gather(b):
            pltpu.make_async_copy(ha_hbm.at[pl.ds(0, CH)], rows.at[b],
                                  gsems[b]).wait()

        def _wait_scatter(b, j):
            pltpu.make_async_copy(rows.at[b], acc.at[didx.at[j]],
                                  ssems[b]).wait()

        def _outer(t, _):
            for b in range(NB):
                k = t * NB + b
                o = 1 - b
                _wait_gather(b)
                pltpu.async_copy(rows.at[b], acc.at[didx.at[k]], ssems[b],
                                 add=True)

                @pl.when(k >= 1)
                def _():
                    _wait_scatter(o, k)

                @pl.when(k + 1 < SR)
                def _():
                    _gather(k + 1, o, gsems[o])
            return 0

        for h in range(halves):
            if h > 0:
                pltpu.sync_copy(srcr_hbm.at[pl.ds(rowbase + h * SR, SR)], sidx)
                pltpu.sync_copy(dstr_hbm.at[pl.ds(rowbase + h * SR, SR)], didx)
            _gather(0, 0, gsems[0])
            if h == 0:
                plsc.subcore_barrier()
            lax.fori_loop(0, SR // NB, _outer, 0)
            _wait_scatter((SR - 1) % NB, 0)

        plsc.subcore_barrier()
        pltpu.sync_copy(acc.at[pl.ds(s * RPT, RPT)],
                        out_hbm.at[c, pl.ds(s * RPT, RPT)])

    return body


def _make_scatter(nch, split_by_wid):
    return pl.kernel(
        _make_scatter_body(nch, split_by_wid),
        out_type=jax.ShapeDtypeStruct((NCORES, NPAD, D), jnp.float32),
        mesh=_mesh,
        scratch_types=[
            pltpu.VMEM((SR, CH), jnp.int32),
            pltpu.VMEM((SR, CH), jnp.int32),
            pltpu.VMEM((NB, CH, D), jnp.float32),
            pltpu.VMEM_SHARED((NPAD, D), jnp.float32),
            pltpu.SemaphoreType.DMA,
            pltpu.SemaphoreType.DMA,
            pltpu.SemaphoreType.DMA,
            pltpu.SemaphoreType.DMA,
            pltpu.SemaphoreType.DMA,
        ],
    )


_scatter1 = _make_scatter(NCH1, True)
_scatter2 = _make_scatter(NCH2, False)


def _head_body(g_hbm, m0_hbm, m1_hbm, out_hbm, gv, m0v, m1v, ov):
    c = lax.axis_index("c")
    s = lax.axis_index("s")
    wid = s * NCORES + c
    base = wid * OPW
    pltpu.sync_copy(g_hbm, gv)
    pltpu.sync_copy(m0_hbm.at[pl.ds(base, OPW)], m0v)
    pltpu.sync_copy(m1_hbm.at[pl.ds(base, OPW)], m1v)

    def _grp(k, _):
        i0 = m0v[pl.ds(k * 16, 16)]
        i1 = m1v[pl.ds(k * 16, 16)]
        v0 = plsc.load_gather(gv, [i0 * 2])
        v1 = plsc.load_gather(gv, [i1 * 2 + 1])
        x = v0 + v1
        ov[pl.ds(k * 16, 16)] = 1.0 / (1.0 + jnp.exp(-x))
        return 0

    lax.fori_loop(0, OPW // 16, _grp, 0)
    pltpu.sync_copy(ov, out_hbm.at[pl.ds(base, OPW)])


_head = pl.kernel(
    _head_body,
    out_type=jax.ShapeDtypeStruct((M,), jnp.float32),
    mesh=_mesh,
    compiler_params=pltpu.CompilerParams(needs_layout_passes=False),
    scratch_types=[
        pltpu.VMEM((2 * N,), jnp.float32),
        pltpu.VMEM((OPW,), jnp.int32),
        pltpu.VMEM((OPW,), jnp.int32),
        pltpu.VMEM((OPW,), jnp.float32),
    ],
)

BLK = 1000


def _mm1_body(p_ref, xp_ref, wrel_ref, wroot_ref, ba_ref, ha_ref, hb_ref):
    agg = p_ref[0] + p_ref[1]
    hp = jnp.dot(agg, wrel_ref[...], preferred_element_type=jnp.float32)
    hp = hp + jnp.dot(xp_ref[...], wroot_ref[...],
                      preferred_element_type=jnp.float32)
    hp = jnp.maximum(hp + ba_ref[...], 0.0)
    ha_ref[...] = hp[:, :D]
    hb_ref[...] = hp[:, D:]


def _mm1(P, xp, wrel, wroot, ba):
    grid = (N // BLK,)
    full = lambda i: (0, 0)
    return pl.pallas_call(
        _mm1_body,
        grid=grid,
        in_specs=[
            pl.BlockSpec((NCORES, BLK, D), lambda i: (0, i, 0)),
            pl.BlockSpec((BLK, D), lambda i: (i, 0)),
            pl.BlockSpec((D, H), full),
            pl.BlockSpec((D, H), full),
            pl.BlockSpec((1, H), full),
        ],
        out_specs=[
            pl.BlockSpec((BLK, D), lambda i: (i, 0)),
            pl.BlockSpec((BLK, D), lambda i: (i, 0)),
        ],
        out_shape=[
            jax.ShapeDtypeStruct((N, D), jnp.float32),
            jax.ShapeDtypeStruct((N, D), jnp.float32),
        ],
    )(P, xp, wrel, wroot, ba)


def _mm2_body(q_ref, ha_ref, hb_ref, wra_ref, wrb_ref, wta_ref, wtb_ref,
              bb_ref, wl_ref, gb_ref, g_ref):
    h2 = jnp.dot(q_ref[0], wra_ref[...], preferred_element_type=jnp.float32)
    h2 = h2 + jnp.dot(q_ref[1], wrb_ref[...], preferred_element_type=jnp.float32)
    h2 = h2 + jnp.dot(ha_ref[...], wta_ref[...], preferred_element_type=jnp.float32)
    h2 = h2 + jnp.dot(hb_ref[...], wtb_ref[...], preferred_element_type=jnp.float32)
    h2 = h2 + bb_ref[...]
    g_ref[...] = jnp.dot(h2, wl_ref[...],
                         preferred_element_type=jnp.float32) + gb_ref[...]


def _mm2(Q, ha, hb, wra, wrb, wta, wtb, bb, wl, gb):
    grid = (N // BLK,)
    full = lambda i: (0, 0)
    return pl.pallas_call(
        _mm2_body,
        grid=grid,
        in_specs=[
            pl.BlockSpec((NCORES, BLK, D), lambda i: (0, i, 0)),
            pl.BlockSpec((BLK, D), lambda i: (i, 0)),
            pl.BlockSpec((BLK, D), lambda i: (i, 0)),
            pl.BlockSpec((D, D), full),
            pl.BlockSpec((D, D), full),
            pl.BlockSpec((D, D), full),
            pl.BlockSpec((D, D), full),
            pl.BlockSpec((1, D), full),
            pl.BlockSpec((D, 2), full),
            pl.BlockSpec((1, 2), full),
        ],
        out_specs=pl.BlockSpec((BLK, 2), lambda i: (i, 0)),
        out_shape=jax.ShapeDtypeStruct((N, 2), jnp.float32),
    )(Q, ha, hb, wra, wrb, wta, wtb, bb, wl, gb)


def kernel(x_protein, x_class, ei_pos, ei_neg, ei_link, ei_ppi, mask,
           W_a_pos_rel, b_a_pos, W_a_pos_root,
           W_a_neg_rel, b_a_neg, W_a_neg_root,
           W_a_link_rel, b_a_link, W_a_link_root,
           W_a_ppi_rel, b_a_ppi, W_a_ppi_root,
           W_b_pos_rel, b_b_pos, W_b_pos_root,
           W_b_neg_rel, b_b_neg, W_b_neg_root,
           W_b_link_rel, b_b_link, W_b_link_root,
           W_b_ppi_rel, b_b_ppi, W_b_ppi_root,
           W_lin, b_lin):
    npd = EP - E
    fill = jnp.arange(npd, dtype=jnp.int32)
    src = jnp.concatenate([ei_ppi[0], (fill * 7) % N]).reshape(ECH, CH)
    dst = jnp.concatenate([ei_ppi[1], N + (fill % (NPAD - N))]).reshape(ECH, CH)
    P = _scatter1(x_protein, x_protein, src, dst)
    ha, hb = _mm1(P, x_protein, W_a_ppi_rel, W_a_ppi_root,
                  b_a_ppi.reshape(1, H))
    Q = _scatter2(ha, hb, src, dst)
    wl = jnp.concatenate([W_lin[:D], W_lin[D:]], axis=1)
    gb = jnp.stack([b_lin[0], jnp.float32(0.0)]).reshape(1, 2)
    G = _mm2(Q, ha, hb, W_b_ppi_rel[:D], W_b_ppi_rel[D:],
             W_b_ppi_root[:D], W_b_ppi_root[D:],
             b_b_ppi.reshape(1, D), wl, gb)
    mt = mask.T
    out = _head(G.reshape(-1), mt[0], mt[1])
    return out.reshape(M, 1)


# speedup vs baseline: 1.1431x; 1.1431x over previous
import jax
import jax.numpy as jnp
from jax import lax
from jax.experimental import pallas as pl
from jax.experimental.pallas import tpu as pltpu
from jax.experimental.pallas import tpu_sc as plsc

N = 10000
D = 128
H = 256
E = 160000
M = 16384
NCORES = 2
NSUB = 16
NW = NCORES * NSUB
CH = 128
EP = 163840
ECH = EP // CH
NCH1 = ECH // NW
NCH2 = ECH // NSUB
NB = 2
SR = 40
NPAD = 10240
RPT = NPAD // NSUB
OPW = M // NW

_mesh = plsc.VectorSubcoreMesh(core_axis_name="c", subcore_axis_name="s")


def _make_scatter_body(nch, split_by_wid):
    halves = nch // SR

    def body(ha_hbm, hb_hbm, srcr_hbm, dstr_hbm, out_hbm,
             sidx, didx, rows, acc, isem, g0, g1):
        c = lax.axis_index("c")
        s = lax.axis_index("s")
        wid = s * NCORES + c
        rowbase = (wid if split_by_wid else s) * nch

        d_si = pltpu.async_copy(srcr_hbm.at[pl.ds(rowbase, SR)], sidx, isem)
        d_di = pltpu.async_copy(dstr_hbm.at[pl.ds(rowbase, SR)], didx, isem)

        z16 = jnp.zeros((16,), jnp.float32)

        def _zrow(r, _):
            def _zcol(cc, _):
                rows[0, r, pl.ds(cc * 16, 16)] = z16
                return 0
            return lax.fori_loop(0, D // 16, _zcol, 0)

        lax.fori_loop(0, CH, _zrow, 0)

        def _zacc(j, _):
            pltpu.sync_copy(rows.at[0], acc.at[pl.ds(s * RPT + j * CH, CH)])
            return 0

        lax.fori_loop(0, RPT // CH, _zacc, 0)
        d_si.wait()
        d_di.wait()

        gsems = (g0, g1)

        def _gather(j, b, sem):
            srow = sidx.at[j]

            @pl.when(c == 0)
            def _():
                pltpu.async_copy(ha_hbm.at[srow], rows.at[b], sem)

            @pl.when(c == 1)
            def _():
                pltpu.async_copy(hb_hbm.at[srow], rows.at[b], sem)

        def _wait_gather(b):
            pltpu.make_async_copy(ha_hbm.at[pl.ds(0, CH)], rows.at[b],
                                  gsems[b]).wait()

        def _outer(t, _):
            for b in range(NB):
                j = t * NB + b
                _wait_gather(b)
                pltpu.sync_copy(rows.at[b], acc.at[didx.at[j]], add=True)

                @pl.when(j + NB < SR)
                def _():
                    _gather(j + NB, b, gsems[b])
            return 0

        for h in range(halves):
            if h > 0:
                pltpu.sync_copy(srcr_hbm.at[pl.ds(rowbase + h * SR, SR)], sidx)
                pltpu.sync_copy(dstr_hbm.at[pl.ds(rowbase + h * SR, SR)], didx)
            for b in range(NB):
                _gather(b, b, gsems[b])
            if h == 0:
                plsc.subcore_barrier()
            lax.fori_loop(0, SR // NB, _outer, 0)

        plsc.subcore_barrier()
        pltpu.sync_copy(acc.at[pl.ds(s * RPT, RPT)],
                        out_hbm.at[c, pl.ds(s * RPT, RPT)])

    return body


def _make_scatter(nch, split_by_wid):
    return pl.kernel(
        _make_scatter_body(nch, split_by_wid),
        out_type=jax.ShapeDtypeStruct((NCORES, NPAD, D), jnp.float32),
        mesh=_mesh,
        scratch_types=[
            pltpu.VMEM((SR, CH), jnp.int32),
            pltpu.VMEM((SR, CH), jnp.int32),
            pltpu.VMEM((NB, CH, D), jnp.float32),
            pltpu.VMEM_SHARED((NPAD, D), jnp.float32),
            pltpu.SemaphoreType.DMA,
            pltpu.SemaphoreType.DMA,
            pltpu.SemaphoreType.DMA,
        ],
    )


_scatter1 = _make_scatter(NCH1, True)
_scatter2 = _make_scatter(NCH2, False)


def _head_body(g_hbm, m0_hbm, m1_hbm, out_hbm, gv, m0v, m1v, ov):
    c = lax.axis_index("c")
    s = lax.axis_index("s")
    wid = s * NCORES + c
    base = wid * OPW
    pltpu.sync_copy(g_hbm, gv)
    pltpu.sync_copy(m0_hbm.at[pl.ds(base, OPW)], m0v)
    pltpu.sync_copy(m1_hbm.at[pl.ds(base, OPW)], m1v)

    def _grp(k, _):
        i0 = m0v[pl.ds(k * 16, 16)]
        i1 = m1v[pl.ds(k * 16, 16)]
        v0 = plsc.load_gather(gv, [i0 * 2])
        v1 = plsc.load_gather(gv, [i1 * 2 + 1])
        x = v0 + v1
        ov[pl.ds(k * 16, 16)] = 1.0 / (1.0 + jnp.exp(-x))
        return 0

    lax.fori_loop(0, OPW // 16, _grp, 0)
    pltpu.sync_copy(ov, out_hbm.at[pl.ds(base, OPW)])


_head = pl.kernel(
    _head_body,
    out_type=jax.ShapeDtypeStruct((M,), jnp.float32),
    mesh=_mesh,
    compiler_params=pltpu.CompilerParams(needs_layout_passes=False),
    scratch_types=[
        pltpu.VMEM((2 * N,), jnp.float32),
        pltpu.VMEM((OPW,), jnp.int32),
        pltpu.VMEM((OPW,), jnp.int32),
        pltpu.VMEM((OPW,), jnp.float32),
    ],
)

BLK = 1000


def _mm1_body(p_ref, xp_ref, wrel_ref, wroot_ref, ba_ref, ha_ref, hb_ref):
    agg = p_ref[0] + p_ref[1]
    hp = jnp.dot(agg, wrel_ref[...], preferred_element_type=jnp.float32)
    hp = hp + jnp.dot(xp_ref[...], wroot_ref[...],
                      preferred_element_type=jnp.float32)
    hp = jnp.maximum(hp + ba_ref[...], 0.0)
    ha_ref[...] = hp[:, :D]
    hb_ref[...] = hp[:, D:]


def _mm1(P, xp, wrel, wroot, ba):
    grid = (N // BLK,)
    full = lambda i: (0, 0)
    return pl.pallas_call(
        _mm1_body,
        grid=grid,
        in_specs=[
            pl.BlockSpec((NCORES, BLK, D), lambda i: (0, i, 0)),
            pl.BlockSpec((BLK, D), lambda i: (i, 0)),
            pl.BlockSpec((D, H), full),
            pl.BlockSpec((D, H), full),
            pl.BlockSpec((1, H), full),
        ],
        out_specs=[
            pl.BlockSpec((BLK, D), lambda i: (i, 0)),
            pl.BlockSpec((BLK, D), lambda i: (i, 0)),
        ],
        out_shape=[
            jax.ShapeDtypeStruct((N, D), jnp.float32),
            jax.ShapeDtypeStruct((N, D), jnp.float32),
        ],
    )(P, xp, wrel, wroot, ba)


def _mm2_body(q_ref, ha_ref, hb_ref, wra_ref, wrb_ref, wta_ref, wtb_ref,
              bb_ref, wl_ref, gb_ref, g_ref):
    h2 = jnp.dot(q_ref[0], wra_ref[...], preferred_element_type=jnp.float32)
    h2 = h2 + jnp.dot(q_ref[1], wrb_ref[...], preferred_element_type=jnp.float32)
    h2 = h2 + jnp.dot(ha_ref[...], wta_ref[...], preferred_element_type=jnp.float32)
    h2 = h2 + jnp.dot(hb_ref[...], wtb_ref[...], preferred_element_type=jnp.float32)
    h2 = h2 + bb_ref[...]
    g_ref[...] = jnp.dot(h2, wl_ref[...],
                         preferred_element_type=jnp.float32) + gb_ref[...]


def _mm2(Q, ha, hb, wra, wrb, wta, wtb, bb, wl, gb):
    grid = (N // BLK,)
    full = lambda i: (0, 0)
    return pl.pallas_call(
        _mm2_body,
        grid=grid,
        in_specs=[
            pl.BlockSpec((NCORES, BLK, D), lambda i: (0, i, 0)),
            pl.BlockSpec((BLK, D), lambda i: (i, 0)),
            pl.BlockSpec((BLK, D), lambda i: (i, 0)),
            pl.BlockSpec((D, D), full),
            pl.BlockSpec((D, D), full),
            pl.BlockSpec((D, D), full),
            pl.BlockSpec((D, D), full),
            pl.BlockSpec((1, D), full),
            pl.BlockSpec((D, 2), full),
            pl.BlockSpec((1, 2), full),
        ],
        out_specs=pl.BlockSpec((BLK, 2), lambda i: (i, 0)),
        out_shape=jax.ShapeDtypeStruct((N, 2), jnp.float32),
    )(Q, ha, hb, wra, wrb, wta, wtb, bb, wl, gb)


def kernel(x_protein, x_class, ei_pos, ei_neg, ei_link, ei_ppi, mask,
           W_a_pos_rel, b_a_pos, W_a_pos_root,
           W_a_neg_rel, b_a_neg, W_a_neg_root,
           W_a_link_rel, b_a_link, W_a_link_root,
           W_a_ppi_rel, b_a_ppi, W_a_ppi_root,
           W_b_pos_rel, b_b_pos, W_b_pos_root,
           W_b_neg_rel, b_b_neg, W_b_neg_root,
           W_b_link_rel, b_b_link, W_b_link_root,
           W_b_ppi_rel, b_b_ppi, W_b_ppi_root,
           W_lin, b_lin):
    npd = EP - E
    fill = jnp.arange(npd, dtype=jnp.int32)
    src = jnp.concatenate([ei_ppi[0], (fill * 7) % N]).reshape(ECH, CH)
    dst = jnp.concatenate([ei_ppi[1], N + (fill % (NPAD - N))]).reshape(ECH, CH)
    P = _scatter1(x_protein, x_protein, src, dst)
    ha, hb = _mm1(P, x_protein, W_a_ppi_rel, W_a_ppi_root,
                  b_a_ppi.reshape(1, H))
    Q = _scatter2(ha, hb, src, dst)
    wl = jnp.concatenate([W_lin[:D], W_lin[D:]], axis=1)
    gb = jnp.stack([b_lin[0], jnp.float32(0.0)]).reshape(1, 2)
    G = _mm2(Q, ha, hb, W_b_ppi_rel[:D], W_b_ppi_rel[D:],
             W_b_ppi_root[:D], W_b_ppi_root[D:],
             b_b_ppi.reshape(1, D), wl, gb)
    mt = mask.T
    out = _head(G.reshape(-1), mt[0], mt[1])
    return out.reshape(M, 1)
